# baseline (speedup 1.0000x reference)
import functools

import jax
import jax.numpy as jnp
from jax.experimental import pallas as pl
from jax.experimental.pallas import tpu as pltpu


def _tap_cols(x_f32, H, W, KH, KW):
    HW = H * W
    C_in = x_f32.shape[0]
    xb = x_f32.astype(jnp.bfloat16)
    hw = jax.lax.broadcasted_iota(jnp.int32, (1, HW), 1)
    wcol = jax.lax.rem(hw, W)
    hrow = jax.lax.div(hw, W)
    cH, cW = (KH - 1) // 2, (KW - 1) // 2

    groups = []
    for kh in range(KH):
        dh = kh - cH
        taps = []
        for kw in range(KW):
            dw = kw - cW
            s = dh * W + dw
            t = pltpu.roll(xb, (-s) % HW, axis=1) if s != 0 else xb
            if dh != 0 or dw != 0:
                m = jnp.full((1, HW), True)
                if dh != 0:
                    m = m & (hrow + dh >= 0) & (hrow + dh < H)
                if dw != 0:
                    m = m & (wcol + dw >= 0) & (wcol + dw < W)
                t = jnp.where(m, t, jnp.bfloat16(0))
            taps.append(t)
        groups.append(jnp.concatenate(taps, axis=0))
    return groups


def _conv_stats_kernel(x_ref, w0_ref, w1_ref, w2_ref, stats_ref, *,
                       H, W, KH, KW):
    C_in = x_ref.shape[1]
    groups = _tap_cols(x_ref[0].reshape(C_in, H * W), H, W, KH, KW)
    wrefs = (w0_ref, w1_ref, w2_ref)
    acc = jnp.dot(wrefs[0][...], groups[0],
                  preferred_element_type=jnp.float32)
    for k in range(1, KH):
        acc = acc + jnp.dot(wrefs[k][...], groups[k],
                            preferred_element_type=jnp.float32)
    s = jnp.sum(acc, axis=1)[None, :]
    ss = jnp.sum(acc * acc, axis=1)[None, :]
    stats_ref[0] = jnp.concatenate([s, ss], axis=0)


def _bn_apply_kernel(x_ref, w0_ref, w1_ref, w2_ref, o_ref, *,
                     H, W, KH, KW):
    C_in = x_ref.shape[1]
    HW = H * W
    groups = _tap_cols(x_ref[0].reshape(C_in, HW), H, W, KH, KW)
    groups[KH - 1] = jnp.concatenate(
        [groups[KH - 1], jnp.ones((1, HW), jnp.bfloat16)], axis=0)
    wrefs = (w0_ref, w1_ref, w2_ref)
    acc = jnp.dot(wrefs[0][...], groups[0],
                  preferred_element_type=jnp.float32)
    for k in range(1, KH):
        acc = acc + jnp.dot(wrefs[k][...], groups[k],
                            preferred_element_type=jnp.float32)
    C_out = o_ref.shape[1]
    o_ref[0] = jnp.maximum(acc, 0.0).reshape(C_out, H, W)


def kernel(x_nchw, weight_oihw, bias, gamma, beta):
    del bias
    N, C_in, H, W = x_nchw.shape
    C_out, _, KH, KW = weight_oihw.shape
    HW = H * W
    eps = 1e-5

    w2 = jnp.transpose(weight_oihw, (0, 2, 3, 1)).reshape(C_out, KH * KW * C_in)
    gk = KW * C_in
    wg = [w2[:, k * gk:(k + 1) * gk].astype(jnp.bfloat16) for k in range(KH)]

    cp = pltpu.CompilerParams(dimension_semantics=("parallel",),
                              vmem_limit_bytes=64 * 1024 * 1024)

    stats = pl.pallas_call(
        functools.partial(_conv_stats_kernel, H=H, W=W, KH=KH, KW=KW),
        out_shape=jax.ShapeDtypeStruct((N, 2, C_out), jnp.float32),
        grid=(N,),
        in_specs=[
            pl.BlockSpec((1, C_in, H, W), lambda n: (n, 0, 0, 0)),
            pl.BlockSpec((C_out, gk), lambda n: (0, 0)),
            pl.BlockSpec((C_out, gk), lambda n: (0, 0)),
            pl.BlockSpec((C_out, gk), lambda n: (0, 0)),
        ],
        out_specs=pl.BlockSpec((1, 2, C_out), lambda n: (n, 0, 0)),
        compiler_params=cp,
    )(x_nchw, wg[0], wg[1], wg[2])

    count = jnp.float32(N * HW)
    mean = jnp.sum(stats[:, 0, :], axis=0) / count
    var = jnp.maximum(jnp.sum(stats[:, 1, :], axis=0) / count - mean * mean,
                      0.0)
    inv_std = jax.lax.rsqrt(var + eps)
    g32 = gamma.astype(jnp.float32)
    scale = g32 * inv_std
    shift = beta.astype(jnp.float32) - mean * scale

    w2s = w2 * scale[:, None]
    wgs = [w2s[:, k * gk:(k + 1) * gk] for k in range(KH)]
    wgs[KH - 1] = jnp.concatenate([wgs[KH - 1], shift[:, None]], axis=1)
    wgs = [w.astype(jnp.bfloat16) for w in wgs]

    y = pl.pallas_call(
        functools.partial(_bn_apply_kernel, H=H, W=W, KH=KH, KW=KW),
        out_shape=jax.ShapeDtypeStruct((N, C_out, H, W), jnp.float32),
        grid=(N,),
        in_specs=[
            pl.BlockSpec((1, C_in, H, W), lambda n: (n, 0, 0, 0)),
            pl.BlockSpec((C_out, gk), lambda n: (0, 0)),
            pl.BlockSpec((C_out, gk), lambda n: (0, 0)),
            pl.BlockSpec((C_out, gk + 1), lambda n: (0, 0)),
        ],
        out_specs=pl.BlockSpec((1, C_out, H, W), lambda n: (n, 0, 0, 0)),
        compiler_params=cp,
    )(x_nchw, wgs[0], wgs[1], wgs[2])

    return y


# speedup vs baseline: 2.1176x; 2.1176x over previous
import functools

import jax
import jax.numpy as jnp
from jax.experimental import pallas as pl
from jax.experimental.pallas import tpu as pltpu


def _tap_cols(xb, H, W, KH, KW):
    HW = H * W
    hw = jax.lax.broadcasted_iota(jnp.int32, (1, HW), 1)
    wcol = jax.lax.rem(hw, W)
    hrow = jax.lax.div(hw, W)
    cH, cW = (KH - 1) // 2, (KW - 1) // 2

    groups = []
    for kh in range(KH):
        dh = kh - cH
        taps = []
        for kw in range(KW):
            dw = kw - cW
            s = dh * W + dw
            t = pltpu.roll(xb, (-s) % HW, axis=1) if s != 0 else xb
            if dh != 0 or dw != 0:
                m = jnp.full((1, HW), True)
                if dh != 0:
                    m = m & (hrow + dh >= 0) & (hrow + dh < H)
                if dw != 0:
                    m = m & (wcol + dw >= 0) & (wcol + dw < W)
                t = jnp.where(m, t, jnp.bfloat16(0))
            taps.append(t)
        groups.append(jnp.concatenate(taps, axis=0))
    return groups


def _conv_stats_kernel(x_ref, w0_ref, w1_ref, w2_ref, stats_ref, xb_ref, *,
                       H, W, KH, KW):
    C_in = x_ref.shape[1]
    xb = x_ref[0].astype(jnp.bfloat16).reshape(C_in, H * W)
    xb_ref[0] = xb
    groups = _tap_cols(xb, H, W, KH, KW)
    wrefs = (w0_ref, w1_ref, w2_ref)
    acc = jnp.dot(wrefs[0][...], groups[0],
                  preferred_element_type=jnp.float32)
    for k in range(1, KH):
        acc = acc + jnp.dot(wrefs[k][...], groups[k],
                            preferred_element_type=jnp.float32)
    s = jnp.sum(acc, axis=1)[None, :]
    ss = jnp.sum(acc * acc, axis=1)[None, :]
    stats_ref[0] = jnp.concatenate([s, ss], axis=0)


def _bn_apply_kernel(x_ref, w0_ref, w1_ref, w2_ref, o_ref, *,
                     H, W, KH, KW):
    HW = H * W
    groups = _tap_cols(x_ref[0], H, W, KH, KW)
    groups[KH - 1] = jnp.concatenate(
        [groups[KH - 1], jnp.ones((1, HW), jnp.bfloat16)], axis=0)
    wrefs = (w0_ref, w1_ref, w2_ref)
    acc = jnp.dot(wrefs[0][...], groups[0],
                  preferred_element_type=jnp.float32)
    for k in range(1, KH):
        acc = acc + jnp.dot(wrefs[k][...], groups[k],
                            preferred_element_type=jnp.float32)
    o_ref[0] = jnp.maximum(acc, 0.0).T


def kernel(x_nchw, weight_oihw, bias, gamma, beta):
    del bias
    N, C_in, H, W = x_nchw.shape
    C_out, _, KH, KW = weight_oihw.shape
    HW = H * W
    eps = 1e-5

    w2 = jnp.transpose(weight_oihw, (0, 2, 3, 1)).reshape(C_out, KH * KW * C_in)
    gk = KW * C_in
    wg = [w2[:, k * gk:(k + 1) * gk].astype(jnp.bfloat16) for k in range(KH)]

    cp = pltpu.CompilerParams(dimension_semantics=("parallel",),
                              vmem_limit_bytes=64 * 1024 * 1024)

    stats, xb = pl.pallas_call(
        functools.partial(_conv_stats_kernel, H=H, W=W, KH=KH, KW=KW),
        out_shape=(
            jax.ShapeDtypeStruct((N, 2, C_out), jnp.float32),
            jax.ShapeDtypeStruct((N, C_in, HW), jnp.bfloat16),
        ),
        grid=(N,),
        in_specs=[
            pl.BlockSpec((1, C_in, H, W), lambda n: (n, 0, 0, 0)),
            pl.BlockSpec((C_out, gk), lambda n: (0, 0)),
            pl.BlockSpec((C_out, gk), lambda n: (0, 0)),
            pl.BlockSpec((C_out, gk), lambda n: (0, 0)),
        ],
        out_specs=(
            pl.BlockSpec((1, 2, C_out), lambda n: (n, 0, 0)),
            pl.BlockSpec((1, C_in, HW), lambda n: (n, 0, 0)),
        ),
        compiler_params=cp,
    )(x_nchw, wg[0], wg[1], wg[2])

    count = jnp.float32(N * HW)
    mean = jnp.sum(stats[:, 0, :], axis=0) / count
    var = jnp.maximum(jnp.sum(stats[:, 1, :], axis=0) / count - mean * mean,
                      0.0)
    inv_std = jax.lax.rsqrt(var + eps)
    g32 = gamma.astype(jnp.float32)
    scale = g32 * inv_std
    shift = beta.astype(jnp.float32) - mean * scale

    w2s = w2 * scale[:, None]
    wgs = [w2s[:, k * gk:(k + 1) * gk] for k in range(KH)]
    wgs[KH - 1] = jnp.concatenate([wgs[KH - 1], shift[:, None]], axis=1)
    wgs = [w.astype(jnp.bfloat16) for w in wgs]

    y = pl.pallas_call(
        functools.partial(_bn_apply_kernel, H=H, W=W, KH=KH, KW=KW),
        out_shape=jax.ShapeDtypeStruct((N, HW, C_out), jnp.float32),
        grid=(N,),
        in_specs=[
            pl.BlockSpec((1, C_in, HW), lambda n: (n, 0, 0)),
            pl.BlockSpec((C_out, gk), lambda n: (0, 0)),
            pl.BlockSpec((C_out, gk), lambda n: (0, 0)),
            pl.BlockSpec((C_out, gk + 1), lambda n: (0, 0)),
        ],
        out_specs=pl.BlockSpec((1, HW, C_out), lambda n: (n, 0, 0)),
        compiler_params=cp,
    )(xb, wgs[0], wgs[1], wgs[2])

    return jnp.transpose(y.reshape(N, H, W, C_out), (0, 3, 1, 2))


# speedup vs baseline: 2.1356x; 1.0085x over previous
import functools

import jax
import jax.numpy as jnp
from jax.experimental import pallas as pl
from jax.experimental.pallas import tpu as pltpu


def _tap_cols(xb, H, W, KH, KW):
    HW = H * W
    hw = jax.lax.broadcasted_iota(jnp.int32, (1, HW), 1)
    wcol = jax.lax.rem(hw, W)
    hrow = jax.lax.div(hw, W)
    cH, cW = (KH - 1) // 2, (KW - 1) // 2

    groups = []
    for kh in range(KH):
        dh = kh - cH
        taps = []
        for kw in range(KW):
            dw = kw - cW
            s = dh * W + dw
            t = pltpu.roll(xb, (-s) % HW, axis=1) if s != 0 else xb
            if dh != 0 or dw != 0:
                m = jnp.full((1, HW), True)
                if dh != 0:
                    m = m & (hrow + dh >= 0) & (hrow + dh < H)
                if dw != 0:
                    m = m & (wcol + dw >= 0) & (wcol + dw < W)
                t = jnp.where(m, t, jnp.bfloat16(0))
            taps.append(t)
        groups.append(jnp.concatenate(taps, axis=0))
    return groups


def _conv_stats_kernel(x_ref, w0_ref, w1_ref, w2_ref, stats_ref, xb_ref, *,
                       H, W, KH, KW):
    C_in = x_ref.shape[1]
    xb = x_ref[0].astype(jnp.bfloat16).reshape(C_in, H * W)
    xb_ref[0] = xb
    groups = _tap_cols(xb, H, W, KH, KW)
    wrefs = (w0_ref, w1_ref, w2_ref)
    acc = jnp.dot(wrefs[0][...], groups[0],
                  preferred_element_type=jnp.float32)
    for k in range(1, KH):
        acc = acc + jnp.dot(wrefs[k][...], groups[k],
                            preferred_element_type=jnp.float32)
    s = jnp.sum(acc, axis=1)[None, :]
    ss = jnp.sum(acc * acc, axis=1)[None, :]
    stats_ref[0] = jnp.concatenate([s, ss], axis=0)


def _bn_apply_kernel(x_ref, w0_ref, w1_ref, w2_ref, sc_ref, sh_ref, o_ref, *,
                     H, W, KH, KW):
    groups = _tap_cols(x_ref[0], H, W, KH, KW)
    wrefs = (w0_ref, w1_ref, w2_ref)
    acc = jnp.dot(wrefs[0][...], groups[0],
                  preferred_element_type=jnp.float32)
    for k in range(1, KH):
        acc = acc + jnp.dot(wrefs[k][...], groups[k],
                            preferred_element_type=jnp.float32)
    o_ref[0] = jnp.maximum(acc.T * sc_ref[...] + sh_ref[...], 0.0)


def kernel(x_nchw, weight_oihw, bias, gamma, beta):
    del bias
    N, C_in, H, W = x_nchw.shape
    C_out, _, KH, KW = weight_oihw.shape
    HW = H * W
    eps = 1e-5

    w2 = jnp.transpose(weight_oihw, (0, 2, 3, 1)).reshape(C_out, KH * KW * C_in)
    gk = KW * C_in
    wg = [w2[:, k * gk:(k + 1) * gk].astype(jnp.bfloat16) for k in range(KH)]

    cp = pltpu.CompilerParams(dimension_semantics=("parallel",),
                              vmem_limit_bytes=64 * 1024 * 1024)

    stats, xb = pl.pallas_call(
        functools.partial(_conv_stats_kernel, H=H, W=W, KH=KH, KW=KW),
        out_shape=(
            jax.ShapeDtypeStruct((N, 2, C_out), jnp.float32),
            jax.ShapeDtypeStruct((N, C_in, HW), jnp.bfloat16),
        ),
        grid=(N,),
        in_specs=[
            pl.BlockSpec((1, C_in, H, W), lambda n: (n, 0, 0, 0)),
            pl.BlockSpec((C_out, gk), lambda n: (0, 0)),
            pl.BlockSpec((C_out, gk), lambda n: (0, 0)),
            pl.BlockSpec((C_out, gk), lambda n: (0, 0)),
        ],
        out_specs=(
            pl.BlockSpec((1, 2, C_out), lambda n: (n, 0, 0)),
            pl.BlockSpec((1, C_in, HW), lambda n: (n, 0, 0)),
        ),
        compiler_params=cp,
    )(x_nchw, wg[0], wg[1], wg[2])

    count = jnp.float32(N * HW)
    mean = jnp.sum(stats[:, 0, :], axis=0) / count
    var = jnp.maximum(jnp.sum(stats[:, 1, :], axis=0) / count - mean * mean,
                      0.0)
    inv_std = jax.lax.rsqrt(var + eps)
    g32 = gamma.astype(jnp.float32)
    scale = (g32 * inv_std).reshape(1, C_out)
    shift = (beta.astype(jnp.float32) - mean.reshape(1, C_out) * scale)

    y = pl.pallas_call(
        functools.partial(_bn_apply_kernel, H=H, W=W, KH=KH, KW=KW),
        out_shape=jax.ShapeDtypeStruct((N, HW, C_out), jnp.float32),
        grid=(N,),
        in_specs=[
            pl.BlockSpec((1, C_in, HW), lambda n: (n, 0, 0)),
            pl.BlockSpec((C_out, gk), lambda n: (0, 0)),
            pl.BlockSpec((C_out, gk), lambda n: (0, 0)),
            pl.BlockSpec((C_out, gk), lambda n: (0, 0)),
            pl.BlockSpec((1, C_out), lambda n: (0, 0)),
            pl.BlockSpec((1, C_out), lambda n: (0, 0)),
        ],
        out_specs=pl.BlockSpec((1, HW, C_out), lambda n: (n, 0, 0)),
        compiler_params=cp,
    )(xb, wg[0], wg[1], wg[2], scale, shift)

    return jnp.transpose(y.reshape(N, H, W, C_out), (0, 3, 1, 2))


# speedup vs baseline: 2.2149x; 1.0371x over previous
import functools

import jax
import jax.numpy as jnp
from jax.experimental import pallas as pl
from jax.experimental.pallas import tpu as pltpu


def _tap_cols(xb, H, W, KH, KW):
    HW = H * W
    hw = jax.lax.broadcasted_iota(jnp.int32, (1, HW), 1)
    wcol = jax.lax.rem(hw, W)
    hrow = jax.lax.div(hw, W)
    cH, cW = (KH - 1) // 2, (KW - 1) // 2

    groups = []
    for kh in range(KH):
        dh = kh - cH
        taps = []
        for kw in range(KW):
            dw = kw - cW
            s = dh * W + dw
            t = pltpu.roll(xb, (-s) % HW, axis=1) if s != 0 else xb
            if dh != 0 or dw != 0:
                m = jnp.full((1, HW), True)
                if dh != 0:
                    m = m & (hrow + dh >= 0) & (hrow + dh < H)
                if dw != 0:
                    m = m & (wcol + dw >= 0) & (wcol + dw < W)
                t = jnp.where(m, t, jnp.bfloat16(0))
            taps.append(t)
        groups.append(jnp.concatenate(taps, axis=0))
    return groups


def _conv_acc(wt, groups, KH, gk):
    dn = (((0,), (0,)), ((), ()))
    acc = jax.lax.dot_general(wt[0 * gk:1 * gk], groups[0], dn,
                              preferred_element_type=jnp.float32)
    for k in range(1, KH):
        acc = acc + jax.lax.dot_general(wt[k * gk:(k + 1) * gk], groups[k],
                                        dn,
                                        preferred_element_type=jnp.float32)
    return acc


def _conv_stats_kernel(x_ref, wt_ref, stats_ref, xb_ref, *, H, W, KH, KW):
    C_in = x_ref.shape[1]
    xb = x_ref[0].astype(jnp.bfloat16).reshape(C_in, H * W)
    xb_ref[0] = xb
    groups = _tap_cols(xb, H, W, KH, KW)
    acc = _conv_acc(wt_ref[...].astype(jnp.bfloat16), groups, KH, KW * C_in)
    s = jnp.sum(acc, axis=1)[None, :]
    ss = jnp.sum(acc * acc, axis=1)[None, :]
    stats_ref[0] = jnp.concatenate([s, ss], axis=0)


def _bn_apply_kernel(x_ref, wt_ref, stats_ref, gb_ref, o_ref, *,
                     H, W, KH, KW, count, eps):
    C_in = x_ref.shape[1]
    sums = jnp.sum(stats_ref[...], axis=0)
    mean = sums[0:1, :] * (1.0 / count)
    ex2 = sums[1:2, :] * (1.0 / count)
    var = jnp.maximum(ex2 - mean * mean, 0.0)
    inv_std = jax.lax.rsqrt(var + eps)
    scale = gb_ref[0:1, :] * inv_std
    shift = gb_ref[1:2, :] - mean * scale

    groups = _tap_cols(x_ref[0], H, W, KH, KW)
    acc = _conv_acc(wt_ref[...].astype(jnp.bfloat16), groups, KH, KW * C_in)
    o_ref[0] = jnp.maximum(acc.T * scale + shift, 0.0)


def kernel(x_nchw, weight_oihw, bias, gamma, beta):
    del bias
    N, C_in, H, W = x_nchw.shape
    C_out, _, KH, KW = weight_oihw.shape
    HW = H * W
    K = KH * KW * C_in

    wt = jnp.transpose(weight_oihw, (2, 3, 1, 0)).reshape(K, C_out)
    gb = jnp.concatenate([gamma.reshape(1, C_out), beta.reshape(1, C_out)],
                         axis=0)

    cp = pltpu.CompilerParams(dimension_semantics=("parallel",),
                              vmem_limit_bytes=64 * 1024 * 1024)

    stats, xb = pl.pallas_call(
        functools.partial(_conv_stats_kernel, H=H, W=W, KH=KH, KW=KW),
        out_shape=(
            jax.ShapeDtypeStruct((N, 2, C_out), jnp.float32),
            jax.ShapeDtypeStruct((N, C_in, HW), jnp.bfloat16),
        ),
        grid=(N,),
        in_specs=[
            pl.BlockSpec((1, C_in, H, W), lambda n: (n, 0, 0, 0)),
            pl.BlockSpec((K, C_out), lambda n: (0, 0)),
        ],
        out_specs=(
            pl.BlockSpec((1, 2, C_out), lambda n: (n, 0, 0)),
            pl.BlockSpec((1, C_in, HW), lambda n: (n, 0, 0)),
        ),
        compiler_params=cp,
    )(x_nchw, wt)

    y = pl.pallas_call(
        functools.partial(_bn_apply_kernel, H=H, W=W, KH=KH, KW=KW,
                          count=float(N * HW), eps=1e-5),
        out_shape=jax.ShapeDtypeStruct((N, HW, C_out), jnp.float32),
        grid=(N,),
        in_specs=[
            pl.BlockSpec((1, C_in, HW), lambda n: (n, 0, 0)),
            pl.BlockSpec((K, C_out), lambda n: (0, 0)),
            pl.BlockSpec((N, 2, C_out), lambda n: (0, 0, 0)),
            pl.BlockSpec((2, C_out), lambda n: (0, 0)),
        ],
        out_specs=pl.BlockSpec((1, HW, C_out), lambda n: (n, 0, 0)),
        compiler_params=cp,
    )(xb, wt, stats, gb)

    return jnp.transpose(y.reshape(N, H, W, C_out), (0, 3, 1, 2))


# speedup vs baseline: 2.6806x; 1.2103x over previous
import functools

import jax
import jax.numpy as jnp
from jax.experimental import pallas as pl
from jax.experimental.pallas import tpu as pltpu


def _tap_cols(xb, H, W, KH, KW):
    HW = H * W
    hw = jax.lax.broadcasted_iota(jnp.int32, (1, HW), 1)
    wcol = jax.lax.rem(hw, W)
    hrow = jax.lax.div(hw, W)
    cH, cW = (KH - 1) // 2, (KW - 1) // 2

    groups = []
    for kh in range(KH):
        dh = kh - cH
        taps = []
        for kw in range(KW):
            dw = kw - cW
            s = dh * W + dw
            t = pltpu.roll(xb, (-s) % HW, axis=1) if s != 0 else xb
            if dh != 0 or dw != 0:
                m = jnp.full((1, HW), True)
                if dh != 0:
                    m = m & (hrow + dh >= 0) & (hrow + dh < H)
                if dw != 0:
                    m = m & (wcol + dw >= 0) & (wcol + dw < W)
                t = jnp.where(m, t, jnp.bfloat16(0))
            taps.append(t)
        groups.append(jnp.concatenate(taps, axis=0))
    return groups


def _conv_acc(wt, groups, KH, gk):
    dn = (((0,), (0,)), ((), ()))
    acc = jax.lax.dot_general(wt[0 * gk:1 * gk], groups[0], dn,
                              preferred_element_type=jnp.float32)
    for k in range(1, KH):
        acc = acc + jax.lax.dot_general(wt[k * gk:(k + 1) * gk], groups[k],
                                        dn,
                                        preferred_element_type=jnp.float32)
    return acc


def _conv_stats_kernel(x_ref, wt_ref, stats_ref, xb_ref, *, H, W, KH, KW):
    B, C_in = x_ref.shape[0], x_ref.shape[1]
    wb = wt_ref[...].astype(jnp.bfloat16)
    s = ss = 0.0
    for b in range(B):
        xb = x_ref[b].astype(jnp.bfloat16).reshape(C_in, H * W)
        xb_ref[b] = xb
        groups = _tap_cols(xb, H, W, KH, KW)
        acc = _conv_acc(wb, groups, KH, KW * C_in)
        s = s + jnp.sum(acc, axis=1)[None, :]
        ss = ss + jnp.sum(acc * acc, axis=1)[None, :]
    stats_ref[0] = jnp.concatenate([s, ss], axis=0)


def _bn_apply_kernel(x_ref, wt_ref, stats_ref, g_ref, b_ref, o_ref, *,
                     H, W, KH, KW, count, eps):
    B = x_ref.shape[0]
    sums = jnp.sum(stats_ref[...], axis=0)
    mean = sums[0:1, :] * (1.0 / count)
    ex2 = sums[1:2, :] * (1.0 / count)
    var = jnp.maximum(ex2 - mean * mean, 0.0)
    inv_std = jax.lax.rsqrt(var + eps)
    scale = g_ref[...].reshape(1, -1) * inv_std
    shift = b_ref[...].reshape(1, -1) - mean * scale

    wb = wt_ref[...].astype(jnp.bfloat16)
    for b in range(B):
        groups = _tap_cols(x_ref[b], H, W, KH, KW)
        acc = _conv_acc(wb, groups, KH, KW * x_ref.shape[1])
        o_ref[b] = jnp.maximum(acc.T * scale + shift, 0.0)


def kernel(x_nchw, weight_oihw, bias, gamma, beta):
    del bias
    N, C_in, H, W = x_nchw.shape
    C_out, _, KH, KW = weight_oihw.shape
    HW = H * W
    K = KH * KW * C_in

    wt = jnp.transpose(weight_oihw, (2, 3, 1, 0)).reshape(K, C_out)
    cp = pltpu.CompilerParams(dimension_semantics=("parallel",),
                              vmem_limit_bytes=100 * 1024 * 1024)

    B = 4
    while N % B:
        B //= 2
    nsteps = N // B

    stats, xb = pl.pallas_call(
        functools.partial(_conv_stats_kernel, H=H, W=W, KH=KH, KW=KW),
        out_shape=(
            jax.ShapeDtypeStruct((nsteps, 2, C_out), jnp.float32),
            jax.ShapeDtypeStruct((N, C_in, HW), jnp.bfloat16),
        ),
        grid=(nsteps,),
        in_specs=[
            pl.BlockSpec((B, C_in, H, W), lambda n: (n, 0, 0, 0)),
            pl.BlockSpec((K, C_out), lambda n: (0, 0)),
        ],
        out_specs=(
            pl.BlockSpec((1, 2, C_out), lambda n: (n, 0, 0)),
            pl.BlockSpec((B, C_in, HW), lambda n: (n, 0, 0)),
        ),
        compiler_params=cp,
    )(x_nchw, wt)

    y = pl.pallas_call(
        functools.partial(_bn_apply_kernel, H=H, W=W, KH=KH, KW=KW,
                          count=float(N * HW), eps=1e-5),
        out_shape=jax.ShapeDtypeStruct((N, HW, C_out), jnp.float32),
        grid=(nsteps,),
        in_specs=[
            pl.BlockSpec((B, C_in, HW), lambda n: (n, 0, 0)),
            pl.BlockSpec((K, C_out), lambda n: (0, 0)),
            pl.BlockSpec((nsteps, 2, C_out), lambda n: (0, 0, 0)),
            pl.BlockSpec((C_out,), lambda n: (0,)),
            pl.BlockSpec((C_out,), lambda n: (0,)),
        ],
        out_specs=pl.BlockSpec((B, HW, C_out), lambda n: (n, 0, 0)),
        compiler_params=cp,
    )(xb, wt, stats, gamma, beta)

    return jnp.transpose(y.reshape(N, H, W, C_out), (0, 3, 1, 2))
